# Initial kernel scaffold; baseline (speedup 1.0000x reference)
import jax
import jax.numpy as jnp
from jax.experimental import pallas as pl


def kernel(input_ids, word_embeddings):
    raise NotImplementedError("write your pallas kernel here")


# speedup vs baseline: 1.6331x; 1.6331x over previous
import functools

import jax
import jax.numpy as jnp
from jax import lax
from jax.experimental import pallas as pl
from jax.experimental.pallas import tpu as pltpu
from jax.experimental.pallas import tpu_sc as plsc

_B = 4 * 4096
_D = 1024
_NC = 2
_NS = 16
_NW = _NC * _NS
_BPW = _B // _NW
_C = 32
_NCHUNK = _BPW // _C


def _emb_body(idx_hbm, table_hbm, out_hbm, idx_v, buf0, buf1,
              gsem0, gsem1, osem0, osem1):
    wid = lax.axis_index("s") * _NC + lax.axis_index("c")
    base = wid * _BPW
    pltpu.sync_copy(idx_hbm.at[pl.ds(base, _BPW)], idx_v)

    bufs = (buf0, buf1)
    gsems = (gsem0, gsem1)
    osems = (osem0, osem1)
    ghandles = [None] * _NCHUNK
    ohandles = [None] * _NCHUNK
    for c in range(_NCHUNK):
        cur = c % 2
        if c >= 2:
            ohandles[c - 2].wait()
        ghandles[c] = pltpu.async_copy(
            table_hbm.at[idx_v.at[pl.ds(c * _C, _C)]], bufs[cur], gsems[cur])
        if c >= 1:
            prev = (c - 1) % 2
            ghandles[c - 1].wait()
            ohandles[c - 1] = pltpu.async_copy(
                bufs[prev], out_hbm.at[pl.ds(base + (c - 1) * _C, _C)],
                osems[prev])
    last = _NCHUNK - 1
    ghandles[last].wait()
    ohandles[last] = pltpu.async_copy(
        bufs[last % 2], out_hbm.at[pl.ds(base + last * _C, _C)],
        osems[last % 2])
    ohandles[last - 1].wait()
    ohandles[last].wait()


@functools.partial(jax.jit, static_argnames=())
def kernel(input_ids, word_embeddings):
    idx = input_ids.reshape(-1).astype(jnp.int32)
    mesh = plsc.VectorSubcoreMesh(core_axis_name="c", subcore_axis_name="s")
    run = pl.kernel(
        _emb_body,
        out_type=jax.ShapeDtypeStruct((_B, _D), jnp.float32),
        mesh=mesh,
        scratch_types=[
            pltpu.VMEM((_BPW,), jnp.int32),
            pltpu.VMEM((_C, _D), jnp.float32),
            pltpu.VMEM((_C, _D), jnp.float32),
            pltpu.SemaphoreType.DMA,
            pltpu.SemaphoreType.DMA,
            pltpu.SemaphoreType.DMA,
            pltpu.SemaphoreType.DMA,
        ],
    )
    out = run(idx, word_embeddings)
    return out.reshape(input_ids.shape + (_D,))
